# baseline (speedup 1.0000x reference)
import functools
import math

import jax
import jax.numpy as jnp
from jax import lax
from jax.experimental import pallas as pl
from jax.experimental.pallas import tpu as pltpu
from jax.experimental.pallas import tpu_sc as plsc

_N = 32768
_K = math.ceil(0.1 * _N)
_ROWS = 64
_NTILES = 32
_RPT = _ROWS // _NTILES
_CHUNKS = _N // 16

_mesh = plsc.VectorSubcoreMesh(core_axis_name="c", subcore_axis_name="s")


def _sc_body(x_hbm, out_hbm, data_v, res_v):
    wid = lax.axis_index("s") * 2 + lax.axis_index("c")
    base = wid * _RPT
    pltpu.sync_copy(x_hbm.at[pl.ds(base, _RPT)], data_v)

    res = jnp.zeros((16,), jnp.float32)
    lane = lax.iota(jnp.int32, 16)

    for r in range(_RPT):
        @plsc.parallel_loop(0, _N, step=16, unroll=8,
                            carry=jnp.zeros((16,), jnp.float32))
        def tot_vec(i, tot):
            v = jnp.maximum(data_v[r, pl.ds(i, 16)], 0.0)
            data_v[r, pl.ds(i, 16)] = v
            return tot + v

        tot = jnp.sum(tot_vec)

        def bs_step(_, carry):
            lo, hi = carry
            w = hi - lo
            m1 = lo + (w >> 2)
            m2 = lo + (w >> 1)
            m3 = lo + (w >> 1) + (w >> 2)

            zeros_i = jnp.zeros((16,), jnp.int32)

            @plsc.parallel_loop(0, _N, step=16, unroll=8,
                                carry=(zeros_i, zeros_i, zeros_i))
            def cnts(i, acc):
                a1, a2, a3 = acc
                b = plsc.bitcast(data_v[r, pl.ds(i, 16)], jnp.int32)
                return (a1 + jnp.where(b >= m1, 1, 0),
                        a2 + jnp.where(b >= m2, 1, 0),
                        a3 + jnp.where(b >= m3, 1, 0))

            ge1 = jnp.sum(cnts[0]) >= _K
            ge2 = jnp.sum(cnts[1]) >= _K
            ge3 = jnp.sum(cnts[2]) >= _K
            new_lo = jnp.where(ge3, m3, jnp.where(ge2, m2, jnp.where(ge1, m1, lo)))
            new_hi = jnp.where(ge1, jnp.where(ge2, jnp.where(ge3, hi, m3), m2), m1)
            return new_lo, new_hi

        lo, _hi = lax.fori_loop(
            0, 16, bs_step, (jnp.int32(0), jnp.int32(0x7F800000))
        )
        t_vec = plsc.bitcast(jnp.full((16,), lo, jnp.int32), jnp.float32)

        @plsc.parallel_loop(0, _N, step=16, unroll=8,
                            carry=(jnp.zeros((16,), jnp.float32),
                                   jnp.zeros((16,), jnp.float32)))
        def sc_pair(i, carry):
            s, c = carry
            v = data_v[r, pl.ds(i, 16)]
            gt = plsc.bitcast(v, jnp.int32) > lo
            return s + jnp.where(gt, v, 0.0), c + jnp.where(gt, 1.0, 0.0)

        s_vec, c_vec = sc_pair
        s = jnp.sum(s_vec)
        c = jnp.sum(c_vec)
        t = t_vec[0]
        topk = s + t * (_K - c)
        cost = topk * (1.0 / _K) - (tot - topk) * (1.0 / (_N - _K))
        res = res + jnp.where(lane == r, cost, 0.0)

    res_v[...] = res
    pltpu.sync_copy(res_v, out_hbm.at[wid])


_sc_call = functools.partial(
    pl.kernel,
    out_type=jax.ShapeDtypeStruct((_NTILES, 16), jnp.float32),
    mesh=_mesh,
    compiler_params=pltpu.CompilerParams(needs_layout_passes=False),
    scratch_types=[
        pltpu.VMEM((_RPT, _N), jnp.float32),
        pltpu.VMEM((16,), jnp.float32),
    ],
)


def _tc_mean_body(p_ref, o_ref):
    o_ref[...] = (jnp.sum(p_ref[...]) / _ROWS).reshape(1, 1)


def kernel(input):
    partials = _sc_call(_sc_body)(input)
    out = pl.pallas_call(
        _tc_mean_body,
        out_shape=jax.ShapeDtypeStruct((1, 1), jnp.float32),
    )(partials)
    return out[0, 0]


# speedup vs baseline: 1.9483x; 1.9483x over previous
import functools
import math

import jax
import jax.numpy as jnp
from jax import lax
from jax.experimental import pallas as pl
from jax.experimental.pallas import tpu as pltpu
from jax.experimental.pallas import tpu_sc as plsc

_N = 32768
_K = math.ceil(0.1 * _N)
_ROWS = 64
_NTILES = 32
_RPT = _ROWS // _NTILES
_CHUNKS = _N // 16

_mesh = plsc.VectorSubcoreMesh(core_axis_name="c", subcore_axis_name="s")


def _sc_body(x_hbm, out_hbm, data_v, res_v):
    wid = lax.axis_index("s") * 2 + lax.axis_index("c")
    base = wid * _RPT
    pltpu.sync_copy(x_hbm.at[pl.ds(base, _RPT)], data_v)

    res = jnp.zeros((16,), jnp.float32)
    lane = lax.iota(jnp.int32, 16)

    for r in range(_RPT):
        @plsc.parallel_loop(0, _N, step=16, unroll=8,
                            carry=jnp.zeros((16,), jnp.float32))
        def tot_vec(i, tot):
            v = jnp.maximum(data_v[r, pl.ds(i, 16)], 0.0)
            data_v[r, pl.ds(i, 16)] = v
            return tot + v

        tot = jnp.sum(tot_vec)

        def bs_step(_, carry):
            lo, hi = carry
            mid = lo + ((hi - lo) >> 1)

            @plsc.parallel_loop(0, _N, step=16, unroll=8,
                                carry=jnp.zeros((16,), jnp.int32))
            def cnt(i, acc):
                b = plsc.bitcast(data_v[r, pl.ds(i, 16)], jnp.int32)
                return acc + jnp.where(b >= mid, 1, 0)

            ge = jnp.sum(cnt) >= _K
            return jnp.where(ge, mid, lo), jnp.where(ge, hi, mid)

        lo, _hi = lax.fori_loop(
            0, 31, bs_step, (jnp.int32(0), jnp.int32(0x7F800000))
        )
        t_vec = plsc.bitcast(jnp.full((16,), lo, jnp.int32), jnp.float32)

        @plsc.parallel_loop(0, _N, step=16, unroll=8,
                            carry=(jnp.zeros((16,), jnp.float32),
                                   jnp.zeros((16,), jnp.float32)))
        def sc_pair(i, carry):
            s, c = carry
            v = data_v[r, pl.ds(i, 16)]
            gt = plsc.bitcast(v, jnp.int32) > lo
            return s + jnp.where(gt, v, 0.0), c + jnp.where(gt, 1.0, 0.0)

        s_vec, c_vec = sc_pair
        s = jnp.sum(s_vec)
        c = jnp.sum(c_vec)
        t = t_vec[0]
        topk = s + t * (_K - c)
        cost = topk * (1.0 / _K) - (tot - topk) * (1.0 / (_N - _K))
        res = res + jnp.where(lane == r, cost, 0.0)

    res_v[...] = res
    pltpu.sync_copy(res_v, out_hbm.at[wid])


_sc_call = functools.partial(
    pl.kernel,
    out_type=jax.ShapeDtypeStruct((_NTILES, 16), jnp.float32),
    mesh=_mesh,
    compiler_params=pltpu.CompilerParams(needs_layout_passes=False),
    scratch_types=[
        pltpu.VMEM((_RPT, _N), jnp.float32),
        pltpu.VMEM((16,), jnp.float32),
    ],
)


def _tc_mean_body(p_ref, o_ref):
    o_ref[...] = (jnp.sum(p_ref[...]) / _ROWS).reshape(1, 1)


def kernel(input):
    partials = _sc_call(_sc_body)(input)
    out = pl.pallas_call(
        _tc_mean_body,
        out_shape=jax.ShapeDtypeStruct((1, 1), jnp.float32),
    )(partials)
    return out[0, 0]


# speedup vs baseline: 3.1916x; 1.6381x over previous
import functools
import math

import jax
import jax.numpy as jnp
from jax import lax
from jax.experimental import pallas as pl
from jax.experimental.pallas import tpu as pltpu
from jax.experimental.pallas import tpu_sc as plsc

_N = 32768
_K = math.ceil(0.1 * _N)
_ROWS = 64
_NTILES = 32
_SC_ROWS = 32
_RPT = _SC_ROWS // _NTILES
_CHUNKS = _N // 16

_mesh = plsc.VectorSubcoreMesh(core_axis_name="c", subcore_axis_name="s")


def _sc_body(x_hbm, out_hbm, data_v, res_v):
    wid = lax.axis_index("s") * 2 + lax.axis_index("c")
    base = wid * _RPT
    pltpu.sync_copy(x_hbm.at[pl.ds(base, _RPT)], data_v)

    res = jnp.zeros((16,), jnp.float32)
    lane = lax.iota(jnp.int32, 16)

    for r in range(_RPT):
        @plsc.parallel_loop(0, _N, step=16, unroll=8,
                            carry=jnp.zeros((16,), jnp.float32))
        def tot_vec(i, tot):
            v = jnp.maximum(data_v[r, pl.ds(i, 16)], 0.0)
            data_v[r, pl.ds(i, 16)] = v
            return tot + v

        tot = jnp.sum(tot_vec)

        def bs_step(_, carry):
            lo, hi = carry
            mid = lo + ((hi - lo) >> 1)

            @plsc.parallel_loop(0, _N, step=16, unroll=8,
                                carry=jnp.zeros((16,), jnp.int32))
            def cnt(i, acc):
                b = plsc.bitcast(data_v[r, pl.ds(i, 16)], jnp.int32)
                return acc + jnp.where(b >= mid, 1, 0)

            ge = jnp.sum(cnt) >= _K
            return jnp.where(ge, mid, lo), jnp.where(ge, hi, mid)

        lo, _hi = lax.fori_loop(
            0, 31, bs_step, (jnp.int32(0), jnp.int32(0x7F800000))
        )
        t_vec = plsc.bitcast(jnp.full((16,), lo, jnp.int32), jnp.float32)

        @plsc.parallel_loop(0, _N, step=16, unroll=8,
                            carry=(jnp.zeros((16,), jnp.float32),
                                   jnp.zeros((16,), jnp.float32)))
        def sc_pair(i, carry):
            s, c = carry
            v = data_v[r, pl.ds(i, 16)]
            gt = plsc.bitcast(v, jnp.int32) > lo
            return s + jnp.where(gt, v, 0.0), c + jnp.where(gt, 1.0, 0.0)

        s_vec, c_vec = sc_pair
        s = jnp.sum(s_vec)
        c = jnp.sum(c_vec)
        t = t_vec[0]
        topk = s + t * (_K - c)
        cost = topk * (1.0 / _K) - (tot - topk) * (1.0 / (_N - _K))
        res = res + jnp.where(lane == r, cost, 0.0)

    res_v[...] = res
    pltpu.sync_copy(res_v, out_hbm.at[wid])


_sc_call = functools.partial(
    pl.kernel,
    out_type=jax.ShapeDtypeStruct((_NTILES, 16), jnp.float32),
    mesh=_mesh,
    compiler_params=pltpu.CompilerParams(needs_layout_passes=False),
    scratch_types=[
        pltpu.VMEM((_RPT, _N), jnp.float32),
        pltpu.VMEM((16,), jnp.float32),
    ],
)


def _tc_rows_body(x_ref, o_ref):
    n = x_ref.shape[1]
    k = _K
    v = jnp.maximum(x_ref[...], 0.0)
    bits = lax.bitcast_convert_type(v, jnp.int32)

    rows = x_ref.shape[0]
    lo0 = jnp.zeros((rows, 1), jnp.int32)
    hi0 = jnp.full((rows, 1), 0x7F800000, jnp.int32)

    def step(_, carry):
        lo, hi = carry
        mid = lo + ((hi - lo) >> 1)
        cnt = jnp.sum((bits >= mid).astype(jnp.int32), axis=1, keepdims=True)
        ge = cnt >= k
        return jnp.where(ge, mid, lo), jnp.where(ge, hi, mid)

    lo, _hi = lax.fori_loop(0, 31, step, (lo0, hi0))
    t = lax.bitcast_convert_type(lo, jnp.float32)

    gt = bits > lo
    s = jnp.sum(jnp.where(gt, v, 0.0), axis=1, keepdims=True)
    c = jnp.sum(gt.astype(jnp.float32), axis=1, keepdims=True)
    tot = jnp.sum(v, axis=1, keepdims=True)
    topk = s + t * (k - c)
    row = topk * (1.0 / k) - (tot - topk) * (1.0 / (n - k))
    o_ref[...] = jnp.sum(row).reshape(1, 1)


def _fin_body(p_ref, q_ref, o_ref):
    o_ref[...] = ((jnp.sum(p_ref[...]) + q_ref[0, 0]) * (1.0 / _ROWS)).reshape(1, 1)


def kernel(input):
    sc_part = _sc_call(_sc_body)(input)
    tc_part = pl.pallas_call(
        _tc_rows_body,
        grid=(1,),
        in_specs=[pl.BlockSpec((_ROWS - _SC_ROWS, _N), lambda i: (1, 0))],
        out_specs=pl.BlockSpec((1, 1), lambda i: (0, 0)),
        out_shape=jax.ShapeDtypeStruct((1, 1), jnp.float32),
    )(input)
    out = pl.pallas_call(
        _fin_body,
        out_shape=jax.ShapeDtypeStruct((1, 1), jnp.float32),
    )(sc_part, tc_part)
    return out[0, 0]
